# Initial kernel scaffold; baseline (speedup 1.0000x reference)
import jax
import jax.numpy as jnp
from jax.experimental import pallas as pl


def kernel(z, embeddings):
    raise NotImplementedError("write your pallas kernel here")


# speedup vs baseline: 1.0590x; 1.0590x over previous
import functools

import jax
import jax.numpy as jnp
from jax import lax
from jax.experimental import pallas as pl
from jax.experimental.pallas import tpu as pltpu
from jax.experimental.pallas import tpu_sc as plsc

_NUM_CODES = 8192
_DIM = 32
_ROWS = 8192
_BLK = 256
_NB = _ROWS // _BLK


def _tc_body(z_ref, emb_ref, idx_ref, loss_ref):
    i = pl.program_id(0)
    zb = z_ref[...]
    eb = emb_ref[...]
    mm = lax.dot_general(
        zb, eb, dimension_numbers=(((1,), (1,)), ((), ())),
        preferred_element_type=jnp.float32)
    zsq = jnp.sum(zb * zb, axis=1, keepdims=True)
    esq = jnp.sum(eb * eb, axis=1)[None, :]
    dist = zsq + esq - 2.0 * mm
    m = jnp.min(dist, axis=1, keepdims=True)
    iota = lax.broadcasted_iota(jnp.int32, dist.shape, 1)
    big = jnp.int32(jnp.iinfo(jnp.int32).max)
    idx_ref[...] = jnp.min(jnp.where(dist == m, iota, big), axis=1)

    @pl.when(i == 0)
    def _():
        loss_ref[0, 0] = 0.0

    loss_ref[0, 0] += jnp.sum(m)


def _tc_argmin(flat_z, embeddings):
    return pl.pallas_call(
        _tc_body,
        grid=(_NB,),
        in_specs=[
            pl.BlockSpec((_BLK, _DIM), lambda i: (i, 0)),
            pl.BlockSpec((_NUM_CODES, _DIM), lambda i: (0, 0)),
        ],
        out_specs=[
            pl.BlockSpec((_BLK,), lambda i: (i,)),
            pl.BlockSpec(memory_space=pltpu.SMEM),
        ],
        out_shape=[
            jax.ShapeDtypeStruct((_ROWS,), jnp.int32),
            jax.ShapeDtypeStruct((1, 1), jnp.float32),
        ],
    )(flat_z, embeddings)


_NC = 2
_NS = 16
_NW = _NC * _NS
_ROWS_PER_W = _ROWS // _NW
_CHUNK = 128
_NCHUNK = _ROWS_PER_W // _CHUNK

def _sc_gather_body(emb_hbm, idx_hbm, out_hbm, idx_v, rows_v, sem):
    wid = lax.axis_index("s") * _NC + lax.axis_index("c")
    pltpu.sync_copy(idx_hbm.at[pl.ds(wid * _NCHUNK, _NCHUNK)], idx_v)
    descs = [
        pltpu.async_copy(emb_hbm.at[idx_v.at[j]],
                         rows_v.at[pl.ds(j * _CHUNK, _CHUNK)], sem)
        for j in range(_NCHUNK)
    ]
    for d in descs:
        d.wait()
    pltpu.sync_copy(rows_v, out_hbm.at[pl.ds(wid * _ROWS_PER_W, _ROWS_PER_W)])


@functools.cache
def _make_sc_gather():
    mesh = plsc.VectorSubcoreMesh(core_axis_name="c", subcore_axis_name="s")
    return pl.kernel(
        _sc_gather_body,
        out_type=jax.ShapeDtypeStruct((_ROWS, _DIM), jnp.float32),
        mesh=mesh,
        scratch_types=[
            pltpu.VMEM((_NCHUNK, _CHUNK), jnp.int32),
            pltpu.VMEM((_ROWS_PER_W, _DIM), jnp.float32),
            pltpu.SemaphoreType.DMA,
        ],
        compiler_params=pltpu.CompilerParams(use_tc_tiling_on_sc=False),
    )


def kernel(z, embeddings):
    input_shape = z.shape
    flat_z = z.reshape(-1, _DIM)
    idx_flat, loss_sum = _tc_argmin(flat_z, embeddings)
    quant_flat = _make_sc_gather()(embeddings, idx_flat.reshape(-1, _CHUNK))
    quantized_st = quant_flat.reshape(input_shape)
    vq_loss = loss_sum[0, 0] * ((1.0 + 0.25) / flat_z.size)
    encoding_indices = idx_flat.reshape(input_shape[0], *input_shape[2:])
    return (quantized_st, vq_loss, encoding_indices)


# speedup vs baseline: 1.2287x; 1.1603x over previous
import functools

import jax
import jax.numpy as jnp
from jax import lax
from jax.experimental import pallas as pl
from jax.experimental.pallas import tpu as pltpu
from jax.experimental.pallas import tpu_sc as plsc

_NUM_CODES = 8192
_DIM = 32
_ROWS = 8192
_BLK = 512
_NB = _ROWS // _BLK


def _tc_body(z_ref, emb_ref, idx_ref, loss_ref):
    i = pl.program_id(0)
    zb = z_ref[...]
    eb = emb_ref[...]
    mm = lax.dot_general(
        zb, eb, dimension_numbers=(((1,), (1,)), ((), ())),
        preferred_element_type=jnp.float32)
    zsq = jnp.sum(zb * zb, axis=1, keepdims=True)
    esq = jnp.sum(eb * eb, axis=1)[None, :]
    dist = zsq + esq - 2.0 * mm
    m = jnp.min(dist, axis=1, keepdims=True)
    iota = lax.broadcasted_iota(jnp.int32, dist.shape, 1)
    big = jnp.int32(jnp.iinfo(jnp.int32).max)
    idx_ref[...] = jnp.min(jnp.where(dist == m, iota, big), axis=1)

    @pl.when(i == 0)
    def _():
        loss_ref[0, 0] = 0.0

    loss_ref[0, 0] += jnp.sum(m)


def _tc_argmin(flat_z, embeddings):
    return pl.pallas_call(
        _tc_body,
        grid=(_NB,),
        in_specs=[
            pl.BlockSpec((_BLK, _DIM), lambda i: (i, 0)),
            pl.BlockSpec((_NUM_CODES, _DIM), lambda i: (0, 0)),
        ],
        out_specs=[
            pl.BlockSpec((_BLK,), lambda i: (i,)),
            pl.BlockSpec(memory_space=pltpu.SMEM),
        ],
        out_shape=[
            jax.ShapeDtypeStruct((_ROWS,), jnp.int32),
            jax.ShapeDtypeStruct((1, 1), jnp.float32),
        ],
    )(flat_z, embeddings)


_NC = 2
_NS = 16
_NW = _NC * _NS
_ROWS_PER_W = _ROWS // _NW
_CHUNK = 128
_NCHUNK = _ROWS_PER_W // _CHUNK

def _sc_gather_body(emb_hbm, idx_hbm, out_hbm, idx_v, rows_v, sem):
    wid = lax.axis_index("s") * _NC + lax.axis_index("c")
    pltpu.sync_copy(idx_hbm.at[pl.ds(wid * _NCHUNK, _NCHUNK)], idx_v)
    descs = [
        pltpu.async_copy(emb_hbm.at[idx_v.at[j]],
                         rows_v.at[pl.ds(j * _CHUNK, _CHUNK)], sem)
        for j in range(_NCHUNK)
    ]
    for d in descs:
        d.wait()
    pltpu.sync_copy(rows_v, out_hbm.at[pl.ds(wid * _ROWS_PER_W, _ROWS_PER_W)])


@functools.cache
def _make_sc_gather():
    mesh = plsc.VectorSubcoreMesh(core_axis_name="c", subcore_axis_name="s")
    return pl.kernel(
        _sc_gather_body,
        out_type=jax.ShapeDtypeStruct((_ROWS, _DIM), jnp.float32),
        mesh=mesh,
        scratch_types=[
            pltpu.VMEM((_NCHUNK, _CHUNK), jnp.int32),
            pltpu.VMEM((_ROWS_PER_W, _DIM), jnp.float32),
            pltpu.SemaphoreType.DMA,
        ],
        compiler_params=pltpu.CompilerParams(use_tc_tiling_on_sc=False),
    )


def kernel(z, embeddings):
    input_shape = z.shape
    flat_z = z.reshape(-1, _DIM)
    idx_flat, loss_sum = _tc_argmin(flat_z, embeddings)
    quant_flat = _make_sc_gather()(embeddings, idx_flat.reshape(-1, _CHUNK))
    quantized_st = quant_flat.reshape(input_shape)
    vq_loss = loss_sum[0, 0] * ((1.0 + 0.25) / flat_z.size)
    encoding_indices = idx_flat.reshape(input_shape[0], *input_shape[2:])
    return (quantized_st, vq_loss, encoding_indices)


# speedup vs baseline: 1.2944x; 1.0534x over previous
import functools

import jax
import jax.numpy as jnp
from jax import lax
from jax.experimental import pallas as pl
from jax.experimental.pallas import tpu as pltpu
from jax.experimental.pallas import tpu_sc as plsc

_NUM_CODES = 8192
_DIM = 32
_ROWS = 8192
_BLK = 1024
_NB = _ROWS // _BLK


def _tc_body(z_ref, emb_ref, idx_ref, loss_ref):
    i = pl.program_id(0)
    zb = z_ref[...]
    eb = emb_ref[...]
    mm = lax.dot_general(
        zb, eb, dimension_numbers=(((1,), (1,)), ((), ())),
        preferred_element_type=jnp.float32)
    zsq = jnp.sum(zb * zb, axis=1, keepdims=True)
    esq = jnp.sum(eb * eb, axis=1)[None, :]
    dist = zsq + esq - 2.0 * mm
    m = jnp.min(dist, axis=1, keepdims=True)
    iota = lax.broadcasted_iota(jnp.int32, dist.shape, 1)
    big = jnp.int32(jnp.iinfo(jnp.int32).max)
    idx_ref[...] = jnp.min(jnp.where(dist == m, iota, big), axis=1)

    @pl.when(i == 0)
    def _():
        loss_ref[0, 0] = 0.0

    loss_ref[0, 0] += jnp.sum(m)


def _tc_argmin(flat_z, embeddings):
    return pl.pallas_call(
        _tc_body,
        grid=(_NB,),
        in_specs=[
            pl.BlockSpec((_BLK, _DIM), lambda i: (i, 0)),
            pl.BlockSpec((_NUM_CODES, _DIM), lambda i: (0, 0)),
        ],
        out_specs=[
            pl.BlockSpec((_BLK,), lambda i: (i,)),
            pl.BlockSpec(memory_space=pltpu.SMEM),
        ],
        out_shape=[
            jax.ShapeDtypeStruct((_ROWS,), jnp.int32),
            jax.ShapeDtypeStruct((1, 1), jnp.float32),
        ],
    )(flat_z, embeddings)


_NC = 2
_NS = 16
_NW = _NC * _NS
_ROWS_PER_W = _ROWS // _NW
_CHUNK = 128
_NCHUNK = _ROWS_PER_W // _CHUNK

def _sc_gather_body(emb_hbm, idx_hbm, out_hbm, idx_v, rows_v, sem):
    wid = lax.axis_index("s") * _NC + lax.axis_index("c")
    pltpu.sync_copy(idx_hbm.at[pl.ds(wid * _NCHUNK, _NCHUNK)], idx_v)
    descs = [
        pltpu.async_copy(emb_hbm.at[idx_v.at[j]],
                         rows_v.at[pl.ds(j * _CHUNK, _CHUNK)], sem)
        for j in range(_NCHUNK)
    ]
    for d in descs:
        d.wait()
    pltpu.sync_copy(rows_v, out_hbm.at[pl.ds(wid * _ROWS_PER_W, _ROWS_PER_W)])


@functools.cache
def _make_sc_gather():
    mesh = plsc.VectorSubcoreMesh(core_axis_name="c", subcore_axis_name="s")
    return pl.kernel(
        _sc_gather_body,
        out_type=jax.ShapeDtypeStruct((_ROWS, _DIM), jnp.float32),
        mesh=mesh,
        scratch_types=[
            pltpu.VMEM((_NCHUNK, _CHUNK), jnp.int32),
            pltpu.VMEM((_ROWS_PER_W, _DIM), jnp.float32),
            pltpu.SemaphoreType.DMA,
        ],
        compiler_params=pltpu.CompilerParams(use_tc_tiling_on_sc=False),
    )


def kernel(z, embeddings):
    input_shape = z.shape
    flat_z = z.reshape(-1, _DIM)
    idx_flat, loss_sum = _tc_argmin(flat_z, embeddings)
    quant_flat = _make_sc_gather()(embeddings, idx_flat.reshape(-1, _CHUNK))
    quantized_st = quant_flat.reshape(input_shape)
    vq_loss = loss_sum[0, 0] * ((1.0 + 0.25) / flat_z.size)
    encoding_indices = idx_flat.reshape(input_shape[0], *input_shape[2:])
    return (quantized_st, vq_loss, encoding_indices)
